# baseline (speedup 1.0000x reference)
import functools

import jax
import jax.numpy as jnp
import numpy as np
from jax import lax
from jax.experimental import pallas as pl
from jax.experimental.pallas import tpu as pltpu
from jax.experimental.pallas import tpu_sc as plsc

VOCAB_NUM = 100000
SEG_NUM = 2
MAX_SEQ_LEN = 2048
D_MODEL = 768
BATCH = 4

NC = 2
NS = 16
NW = NC * NS
B_TOTAL = BATCH * MAX_SEQ_LEN
B_PER_W = B_TOTAL // NW
W_PER_B = MAX_SEQ_LEN // B_PER_W
CHUNK = 32
NBUF = 2
N_CHUNKS = B_PER_W // CHUNK
LANES = 16
GROUPS = D_MODEL // LANES


def _pos_enc_table():
    pos_ids = np.arange(0, MAX_SEQ_LEN, 1, dtype=np.float32)[:, None]
    div_term = np.power(
        10000.0, np.arange(0, D_MODEL, 2, dtype=np.float32) / D_MODEL)
    pe = np.zeros((MAX_SEQ_LEN, D_MODEL), dtype=np.float32)
    pe[:, ::2] = np.sin(pos_ids / div_term)
    pe[:, 1::2] = np.cos(pos_ids / div_term)
    return pe


_POS_ENC = _pos_enc_table()

_PERM = np.empty(D_MODEL, dtype=np.int64)
for _base in range(0, D_MODEL, 2 * LANES):
    for _j in range(LANES):
        _PERM[_base + 2 * _j] = _base + _j
        _PERM[_base + 2 * _j + 1] = _base + LANES + _j
_POS_ENC_PERM = _POS_ENC[:, _PERM]


@functools.partial(
    pl.kernel,
    out_type=jax.ShapeDtypeStruct((BATCH, MAX_SEQ_LEN, D_MODEL), jnp.float32),
    mesh=plsc.VectorSubcoreMesh(core_axis_name="c", subcore_axis_name="s"),
    scratch_types=[
        [pltpu.VMEM((CHUNK,), jnp.int32)] * NBUF,
        [pltpu.VMEM((CHUNK,), jnp.int32)] * NBUF,
        [pltpu.VMEM((CHUNK, D_MODEL), jnp.float32)] * NBUF,
        [pltpu.VMEM((CHUNK, D_MODEL // 2), jnp.int32)] * NBUF,
        [pltpu.SemaphoreType.DMA] * NBUF,
        [pltpu.SemaphoreType.DMA] * NBUF,
        [pltpu.SemaphoreType.DMA] * NBUF,
    ],
)
def _emb_kernel(ids_hbm, segs_hbm, tok_hbm, comb_hbm, out_hbm,
                idx_t, idx_c, buf_a, buf_b, sem_a, sem_b, sem_o):
    wid = lax.axis_index("s") * NC + lax.axis_index("c")
    b = wid // W_PER_B
    pos_base = (wid % W_PER_B) * B_PER_W

    def issue(k, s):
        p0 = pos_base + k * CHUNK
        pltpu.sync_copy(ids_hbm.at[b, pl.ds(p0, CHUNK)], idx_t[s])
        pltpu.sync_copy(segs_hbm.at[b, pl.ds(p0, CHUNK)], idx_c[s])
        for g in range(CHUNK // LANES):
            sl = pl.ds(g * LANES, LANES)
            seg = idx_c[s][sl]
            iota = lax.iota(jnp.int32, LANES)
            idx_c[s][sl] = seg * MAX_SEQ_LEN + (p0 + g * LANES) + iota
        cp_a = pltpu.async_copy(tok_hbm.at[idx_t[s]], buf_a[s], sem_a[s])
        cp_b = pltpu.async_copy(comb_hbm.at[idx_c[s]], buf_b[s], sem_b[s])
        return cp_a, cp_b

    gathers = [None] * NBUF
    stores = [None] * NBUF
    for k in range(NBUF - 1):
        gathers[k] = issue(k, k)
    for k in range(N_CHUNKS):
        s = k % NBUF
        if k + NBUF - 1 < N_CHUNKS:
            n = (k + NBUF - 1) % NBUF
            if stores[n] is not None:
                stores[n].wait()
            gathers[n] = issue(k + NBUF - 1, n)
        cp_a, cp_b = gathers[s]
        cp_a.wait()
        cp_b.wait()

        def add_row(r, _, s=s):
            for g in range(GROUPS // 2):
                w = buf_b[s][r, pl.ds(g * LANES, LANES)]
                lo = lax.bitcast_convert_type(w << 16, jnp.float32)
                hi = lax.bitcast_convert_type(
                    w & jnp.int32(-65536), jnp.float32)
                plsc.addupdate(buf_a[s].at[r, pl.ds(g * 2 * LANES, LANES)], lo)
                plsc.addupdate(
                    buf_a[s].at[r, pl.ds(g * 2 * LANES + LANES, LANES)], hi)
            return 0

        lax.fori_loop(0, CHUNK, add_row, 0)
        stores[s] = pltpu.async_copy(
            buf_a[s], out_hbm.at[b, pl.ds(pos_base + k * CHUNK, CHUNK)],
            sem_o[s])
    for st in stores:
        if st is not None:
            st.wait()


def kernel(input_ids, seg_ids, masks, token_table, segment_table):
    del masks
    comb = (segment_table[:, _PERM][:, None, :] + _POS_ENC_PERM[None, :, :])
    comb = comb.reshape(SEG_NUM * MAX_SEQ_LEN, D_MODEL).astype(jnp.bfloat16)
    comb = jax.lax.bitcast_convert_type(
        comb.reshape(SEG_NUM * MAX_SEQ_LEN, D_MODEL // 2, 2), jnp.int32)
    return _emb_kernel(input_ids.astype(jnp.int32),
                       seg_ids.astype(jnp.int32), token_table, comb)


# speedup vs baseline: 2.2781x; 2.2781x over previous
import functools

import jax
import jax.numpy as jnp
import numpy as np
from jax import lax
from jax.experimental import pallas as pl
from jax.experimental.pallas import tpu as pltpu
from jax.experimental.pallas import tpu_sc as plsc

VOCAB_NUM = 100000
SEG_NUM = 2
MAX_SEQ_LEN = 2048
D_MODEL = 768
BATCH = 4

NC = 2
NS = 16
NW = NC * NS
B_TOTAL = BATCH * MAX_SEQ_LEN
B_PER_W = B_TOTAL // NW
W_PER_B = MAX_SEQ_LEN // B_PER_W
CHUNK = 32
NBUF = 2
N_CHUNKS = B_PER_W // CHUNK
LANES = 16
GROUPS = D_MODEL // LANES


def _pos_enc_table():
    pos_ids = np.arange(0, MAX_SEQ_LEN, 1, dtype=np.float32)[:, None]
    div_term = np.power(
        10000.0, np.arange(0, D_MODEL, 2, dtype=np.float32) / D_MODEL)
    pe = np.zeros((MAX_SEQ_LEN, D_MODEL), dtype=np.float32)
    pe[:, ::2] = np.sin(pos_ids / div_term)
    pe[:, 1::2] = np.cos(pos_ids / div_term)
    return pe


_POS_ENC = _pos_enc_table()


@functools.partial(
    pl.kernel,
    out_type=jax.ShapeDtypeStruct((BATCH, MAX_SEQ_LEN, D_MODEL), jnp.float32),
    mesh=plsc.VectorSubcoreMesh(core_axis_name="c", subcore_axis_name="s"),
    scratch_types=[
        pltpu.VMEM((B_PER_W,), jnp.int32),
        pltpu.VMEM((B_PER_W,), jnp.int32),
        [pltpu.VMEM((CHUNK, D_MODEL), jnp.float32)] * NBUF,
        [pltpu.VMEM((CHUNK, D_MODEL), jnp.float32)] * NBUF,
        pltpu.SemaphoreType.DMA,
        [pltpu.SemaphoreType.DMA] * NBUF,
        [pltpu.SemaphoreType.DMA] * NBUF,
        [pltpu.SemaphoreType.DMA] * NBUF,
    ],
)
def _emb_kernel(ids_hbm, segs_hbm, tok_hbm, comb_hbm, out_hbm,
                idx_t, idx_c, buf_a, buf_b, sem_i, sem_a, sem_b, sem_o):
    wid = lax.axis_index("s") * NC + lax.axis_index("c")
    b = wid // W_PER_B
    pos_base = (wid % W_PER_B) * B_PER_W

    cp_t = pltpu.async_copy(ids_hbm.at[b, pl.ds(pos_base, B_PER_W)], idx_t,
                            sem_i)
    cp_c = pltpu.async_copy(segs_hbm.at[b, pl.ds(pos_base, B_PER_W)], idx_c,
                            sem_i)
    cp_t.wait()
    cp_c.wait()
    for g in range(B_PER_W // LANES):
        sl = pl.ds(g * LANES, LANES)
        iota = lax.iota(jnp.int32, LANES)
        idx_c[sl] = (idx_c[sl] * MAX_SEQ_LEN
                     + (pos_base + g * LANES) + iota)

    def issue(k, s):
        r0 = pl.ds(k * CHUNK, CHUNK)
        cp_a = pltpu.async_copy(tok_hbm.at[idx_t.at[r0]], buf_a[s], sem_a[s])
        cp_b = pltpu.async_copy(comb_hbm.at[idx_c.at[r0]], buf_b[s], sem_b[s])
        return cp_a, cp_b

    gathers = [None] * NBUF
    stores = [None] * NBUF
    for k in range(NBUF - 1):
        gathers[k] = issue(k, k)
    for k in range(N_CHUNKS):
        s = k % NBUF
        if k + NBUF - 1 < N_CHUNKS:
            n = (k + NBUF - 1) % NBUF
            if stores[n] is not None:
                stores[n].wait()
            gathers[n] = issue(k + NBUF - 1, n)
        cp_a, cp_b = gathers[s]
        cp_a.wait()
        cp_b.wait()

        def add_row(r, _, s=s):
            for g in range(GROUPS):
                sl = pl.ds(g * LANES, LANES)
                plsc.addupdate(buf_a[s].at[r, sl], buf_b[s][r, sl])
            return 0

        lax.fori_loop(0, CHUNK, add_row, 0)
        stores[s] = pltpu.async_copy(
            buf_a[s], out_hbm.at[b, pl.ds(pos_base + k * CHUNK, CHUNK)],
            sem_o[s])
    for st in stores:
        if st is not None:
            st.wait()


def kernel(input_ids, seg_ids, masks, token_table, segment_table):
    del masks
    comb = (segment_table[:, None, :] + _POS_ENC[None, :, :])
    comb = comb.reshape(SEG_NUM * MAX_SEQ_LEN, D_MODEL)
    return _emb_kernel(input_ids.astype(jnp.int32),
                       seg_ids.astype(jnp.int32), token_table, comb)
